# baseline (speedup 1.0000x reference)
import functools

import jax
import jax.numpy as jnp
from jax import lax
from jax.experimental import pallas as pl
from jax.experimental.pallas import tpu as pltpu
from jax.experimental.pallas import tpu_sc as plsc

VOCAB = 400000
EMB_DIM = 100
HIDDEN = 128
NUM_CLASSES = 4
BATCH = 4096
SEQLEN = 200

DP = 112
NC = 2
NS = 16
NW = NC * NS
CHUNK = 100
CP = 104
CPW = (BATCH * SEQLEN) // (NW * CHUNK)
RPW = BATCH // NW
LANES = 16
OUT_OFFS = (0, 16, 32, 48, 64, 80, 84)


def _pool_body(x_hbm, tab_hbm, pooled_hbm, idx_v, buf0, buf1, acc, out_v,
               sem0, sem1):
    cid = lax.axis_index("c")
    sid = lax.axis_index("s")
    wid = sid * NC + cid
    cbase = wid * CPW

    pltpu.sync_copy(x_hbm.at[pl.ds(cbase, CPW)], idx_v)

    zvec = jnp.zeros((LANES,), jnp.float32)

    pltpu.async_copy(tab_hbm.at[idx_v.at[0]], buf0, sem0)
    pltpu.async_copy(tab_hbm.at[idx_v.at[1]], buf1, sem1)

    def accumulate(buf):
        @pl.loop(0, CHUNK // 4)
        def _(g):
            for rr in range(4):
                r = g * 4 + rr
                for v in range(DP // LANES):
                    plsc.addupdate(acc.at[pl.ds(v * LANES, LANES)],
                                   buf[r, pl.ds(v * LANES, LANES)])

    @pl.loop(0, RPW)
    def _(i):
        for v in range(DP // LANES):
            acc[pl.ds(v * LANES, LANES)] = zvec

        c0 = 2 * i
        pltpu.make_async_copy(tab_hbm.at[idx_v.at[c0]], buf0, sem0).wait()
        accumulate(buf0)
        nxt0 = jnp.minimum(c0 + 2, CPW - 2)
        pltpu.async_copy(tab_hbm.at[idx_v.at[nxt0]], buf0, sem0)

        pltpu.make_async_copy(tab_hbm.at[idx_v.at[c0 + 1]], buf1, sem1).wait()
        accumulate(buf1)
        nxt1 = jnp.minimum(c0 + 3, CPW - 1)
        pltpu.async_copy(tab_hbm.at[idx_v.at[nxt1]], buf1, sem1)

        for off in OUT_OFFS:
            out_v[i, pl.ds(off, LANES)] = acc[pl.ds(off, LANES)]

    pltpu.make_async_copy(tab_hbm.at[idx_v.at[CPW - 2]], buf0, sem0).wait()
    pltpu.make_async_copy(tab_hbm.at[idx_v.at[CPW - 1]], buf1, sem1).wait()

    pltpu.sync_copy(out_v, pooled_hbm.at[pl.ds(wid * RPW, RPW)])


@functools.partial(
    pl.kernel,
    out_type=jax.ShapeDtypeStruct((BATCH, EMB_DIM), jnp.float32),
    mesh=plsc.VectorSubcoreMesh(core_axis_name="c", subcore_axis_name="s"),
    compiler_params=pltpu.CompilerParams(use_tc_tiling_on_sc=False),
    scratch_types=[
        pltpu.VMEM((CPW, CP), jnp.int32),
        pltpu.VMEM((CP, DP), jnp.float32),
        pltpu.VMEM((CP, DP), jnp.float32),
        pltpu.VMEM((DP,), jnp.float32),
        pltpu.VMEM((RPW, EMB_DIM), jnp.float32),
        pltpu.SemaphoreType.DMA,
        pltpu.SemaphoreType.DMA,
    ],
)
def _pool(x_hbm, tab_hbm, pooled_hbm, *rest):
    _pool_body(x_hbm, tab_hbm, pooled_hbm, *rest)


def _mlp_body(p_ref, w1_ref, b1_ref, w2_ref, b2_ref, o_ref):
    h = jnp.dot(p_ref[...], w1_ref[...], preferred_element_type=jnp.float32)
    h = h * (1.0 / SEQLEN) + b1_ref[...]
    h = jnp.maximum(h, 0.0)
    o_ref[...] = (
        jnp.dot(h, w2_ref[...], preferred_element_type=jnp.float32)
        + b2_ref[...]
    )


_mlp = pl.pallas_call(
    _mlp_body,
    out_shape=jax.ShapeDtypeStruct((BATCH, NUM_CLASSES), jnp.float32),
)


@jax.jit
def kernel(x, emb_table, W1, b1, W2, b2):
    tabp = jnp.pad(emb_table, ((0, 0), (0, DP - EMB_DIM)))
    xp = jnp.pad(x.reshape(-1, CHUNK), ((0, 0), (0, CP - CHUNK)))
    pooled = _pool(xp, tabp)
    return _mlp(pooled, W1, b1.reshape(1, HIDDEN), W2,
                b2.reshape(1, NUM_CLASSES))


# speedup vs baseline: 1.0011x; 1.0011x over previous
import functools

import jax
import jax.numpy as jnp
from jax import lax
from jax.experimental import pallas as pl
from jax.experimental.pallas import tpu as pltpu
from jax.experimental.pallas import tpu_sc as plsc

VOCAB = 400000
EMB_DIM = 100
HIDDEN = 128
NUM_CLASSES = 4
BATCH = 4096
SEQLEN = 200

DP = 112
NC = 2
NS = 16
NW = NC * NS
CHUNK = 100
CP = 104
CPW = (BATCH * SEQLEN) // (NW * CHUNK)
RPW = BATCH // NW
LANES = 16
OUT_OFFS = (0, 16, 32, 48, 64, 80, 84)


def _pool_body(x_hbm, tab_hbm, pooled_hbm, idx_v, buf0, buf1, acc, out_v,
               sem0, sem1):
    cid = lax.axis_index("c")
    sid = lax.axis_index("s")
    wid = sid * NC + cid
    cbase = wid * CPW

    pltpu.sync_copy(x_hbm.at[pl.ds(cbase, CPW)], idx_v)

    zvec = jnp.zeros((LANES,), jnp.float32)

    pltpu.async_copy(tab_hbm.at[idx_v.at[0]], buf0, sem0)
    pltpu.async_copy(tab_hbm.at[idx_v.at[1]], buf1, sem1)

    def accumulate(buf, carry_in):
        @pl.loop(0, CHUNK // 4, init_carry=carry_in)
        def carry_out(g, carry):
            vs = list(carry)
            for rr in range(4):
                r = g * 4 + rr
                for v in range(DP // LANES):
                    vs[v] = vs[v] + buf[r, pl.ds(v * LANES, LANES)]
            return tuple(vs)

        return carry_out

    @pl.loop(0, RPW)
    def _(i):
        acc7 = (zvec,) * (DP // LANES)

        c0 = 2 * i
        pltpu.make_async_copy(tab_hbm.at[idx_v.at[c0]], buf0, sem0).wait()
        acc7 = accumulate(buf0, acc7)
        nxt0 = jnp.minimum(c0 + 2, CPW - 2)
        pltpu.async_copy(tab_hbm.at[idx_v.at[nxt0]], buf0, sem0)

        pltpu.make_async_copy(tab_hbm.at[idx_v.at[c0 + 1]], buf1, sem1).wait()
        acc7 = accumulate(buf1, acc7)
        nxt1 = jnp.minimum(c0 + 3, CPW - 1)
        pltpu.async_copy(tab_hbm.at[idx_v.at[nxt1]], buf1, sem1)

        for v in range(6):
            out_v[i, pl.ds(v * LANES, LANES)] = acc7[v]
        acc[pl.ds(80, LANES)] = acc7[5]
        acc[pl.ds(96, LANES)] = acc7[6]
        out_v[i, pl.ds(84, LANES)] = acc[pl.ds(84, LANES)]

    pltpu.make_async_copy(tab_hbm.at[idx_v.at[CPW - 2]], buf0, sem0).wait()
    pltpu.make_async_copy(tab_hbm.at[idx_v.at[CPW - 1]], buf1, sem1).wait()

    pltpu.sync_copy(out_v, pooled_hbm.at[pl.ds(wid * RPW, RPW)])


@functools.partial(
    pl.kernel,
    out_type=jax.ShapeDtypeStruct((BATCH, EMB_DIM), jnp.float32),
    mesh=plsc.VectorSubcoreMesh(core_axis_name="c", subcore_axis_name="s"),
    compiler_params=pltpu.CompilerParams(use_tc_tiling_on_sc=False),
    scratch_types=[
        pltpu.VMEM((CPW, CP), jnp.int32),
        pltpu.VMEM((CP, DP), jnp.float32),
        pltpu.VMEM((CP, DP), jnp.float32),
        pltpu.VMEM((DP,), jnp.float32),
        pltpu.VMEM((RPW, EMB_DIM), jnp.float32),
        pltpu.SemaphoreType.DMA,
        pltpu.SemaphoreType.DMA,
    ],
)
def _pool(x_hbm, tab_hbm, pooled_hbm, *rest):
    _pool_body(x_hbm, tab_hbm, pooled_hbm, *rest)


def _mlp_body(p_ref, w1_ref, b1_ref, w2_ref, b2_ref, o_ref):
    h = jnp.dot(p_ref[...], w1_ref[...], preferred_element_type=jnp.float32)
    h = h * (1.0 / SEQLEN) + b1_ref[...]
    h = jnp.maximum(h, 0.0)
    o_ref[...] = (
        jnp.dot(h, w2_ref[...], preferred_element_type=jnp.float32)
        + b2_ref[...]
    )


_mlp = pl.pallas_call(
    _mlp_body,
    out_shape=jax.ShapeDtypeStruct((BATCH, NUM_CLASSES), jnp.float32),
)


@jax.jit
def kernel(x, emb_table, W1, b1, W2, b2):
    tabp = jnp.pad(emb_table, ((0, 0), (0, DP - EMB_DIM)))
    xp = jnp.pad(x.reshape(-1, CHUNK), ((0, 0), (0, CP - CHUNK)))
    pooled = _pool(xp, tabp)
    return _mlp(pooled, W1, b1.reshape(1, HIDDEN), W2,
                b2.reshape(1, NUM_CLASSES))
